# baseline (speedup 1.0000x reference)
import functools

import jax
import jax.numpy as jnp
from jax import lax
from jax.experimental import pallas as pl
from jax.experimental.pallas import tpu as pltpu
from jax.experimental.pallas import tpu_sc as plsc

NC = 2
NS = 16
L = 16
NW = NC * NS

B = 16384
D = 32
BPW = B // NW
CHUNK = 128
NCHUNK = BPW // CHUNK
NGROUP = BPW // L


def _sc_kernel(idx6, event_em, edgetype_em, attrib_em, out_hbm,
               idx_v, ph, pr, pt, nh, nr, nt, out_v, sem):
    wid = lax.axis_index("s") * NC + lax.axis_index("c")
    base = wid * BPW

    pltpu.sync_copy(idx6.at[:, wid], idx_v)

    tables = (event_em, edgetype_em, attrib_em,
              event_em, edgetype_em, attrib_em)
    bufs = (ph, pr, pt, nh, nr, nt)

    copies = []
    for j in range(6):
        for c in range(NCHUNK):
            cp = pltpu.make_async_copy(
                tables[j].at[idx_v.at[j, pl.ds(c * CHUNK, CHUNK)]],
                bufs[j].at[pl.ds(c * CHUNK, CHUNK), :],
                sem,
            )
            cp.start()
            copies.append(cp)
    for cp in copies:
        cp.wait()

    def l1_dist(hb, rb, tb, b):
        s = jnp.abs(hb[b, pl.ds(0, L)] + rb[b, pl.ds(0, L)] - tb[b, pl.ds(0, L)])
        s = s + jnp.abs(
            hb[b, pl.ds(L, L)] + rb[b, pl.ds(L, L)] - tb[b, pl.ds(L, L)]
        )
        return jnp.sum(s)

    lane = lax.iota(jnp.int32, L)
    zeros = jnp.zeros((L,), jnp.float32)

    def group_body(g, _):
        vloss = zeros
        for u in range(L):
            b = g * L + u
            pd = l1_dist(ph, pr, pt, b)
            nd = l1_dist(nh, nr, nt, b)
            sc = pd - nd + jnp.float32(1.0)
            vloss = jnp.where(lane == u, lax.broadcast(sc, (L,)), vloss)
        out_v[pl.ds(g * L, L)] = jnp.maximum(zeros, vloss)
        return 0

    lax.fori_loop(0, NGROUP, group_body, 0)

    pltpu.sync_copy(out_v, out_hbm.at[pl.ds(base, BPW)])


@jax.jit
def _run(idx6, event_em, edgetype_em, attrib_em):
    mesh = plsc.VectorSubcoreMesh(core_axis_name="c", subcore_axis_name="s")
    return pl.kernel(
        _sc_kernel,
        out_type=jax.ShapeDtypeStruct((B,), jnp.float32),
        mesh=mesh,
        compiler_params=pltpu.CompilerParams(
            needs_layout_passes=False, use_tc_tiling_on_sc=False
        ),
        scratch_types=[
            pltpu.VMEM((6, BPW), jnp.int32),
            pltpu.VMEM((BPW, D), jnp.float32),
            pltpu.VMEM((BPW, D), jnp.float32),
            pltpu.VMEM((BPW, D), jnp.float32),
            pltpu.VMEM((BPW, D), jnp.float32),
            pltpu.VMEM((BPW, D), jnp.float32),
            pltpu.VMEM((BPW, D), jnp.float32),
            pltpu.VMEM((BPW,), jnp.float32),
            pltpu.SemaphoreType.DMA,
        ],
    )(idx6, event_em, edgetype_em, attrib_em)


def kernel(pos_triplets, neg_triplets, event_em, edgetype_em, attrib_em):
    idx6 = jnp.concatenate(
        [pos_triplets.T, neg_triplets.T], axis=0
    ).reshape(6, NW, BPW)
    return _run(idx6, event_em, edgetype_em, attrib_em)
